# baseline (speedup 1.0000x reference)
import functools

import jax
import jax.numpy as jnp
from jax import lax
from jax.experimental import pallas as pl
from jax.experimental.pallas import tpu as pltpu
from jax.experimental.pallas import tpu_sc as plsc

VOCAB_SIZE = 1000000
EMBED_DIM = 16
BATCH = 16384
MAX_NORM = 2.0

ROWS_PER_TILE = 128 // EMBED_DIM
TAB_ROWS = VOCAB_SIZE // ROWS_PER_TILE

NUM_CORES = 2
NUM_SUBCORES = 16
NUM_WORKERS = NUM_CORES * NUM_SUBCORES
ROWS_PER_WORKER = BATCH // NUM_WORKERS
CHUNK = 128
CHUNKS_PER_WORKER = ROWS_PER_WORKER // CHUNK
BLOCKS_PER_WORKER = ROWS_PER_WORKER // 16
OUT_STAGE_ROWS = ROWS_PER_WORKER // ROWS_PER_TILE


def _rsqrt(x):
    i = lax.bitcast_convert_type(x, jnp.int32)
    y = lax.bitcast_convert_type(jnp.int32(0x5F3759DF) - (i >> 1), jnp.float32)
    for _ in range(3):
        y = y * (1.5 - 0.5 * x * y * y)
    return y


def _sc_embed(table2, idx2d):
    mesh = plsc.VectorSubcoreMesh(core_axis_name="c", subcore_axis_name="s")

    @functools.partial(
        pl.kernel,
        out_type=jax.ShapeDtypeStruct((BATCH // ROWS_PER_TILE, 128), jnp.float32),
        mesh=mesh,
        compiler_params=pltpu.CompilerParams(needs_layout_passes=False),
        scratch_types=[
            pltpu.VMEM((CHUNKS_PER_WORKER, CHUNK), jnp.int32),
            pltpu.VMEM((CHUNKS_PER_WORKER, CHUNK), jnp.int32),
            pltpu.VMEM((ROWS_PER_WORKER,), jnp.int32),
            pltpu.VMEM((ROWS_PER_WORKER, 128), jnp.float32),
            pltpu.VMEM((OUT_STAGE_ROWS, 128), jnp.float32),
            pltpu.SemaphoreType.DMA,
        ],
    )
    def k(table_hbm, idx_hbm, out_hbm, idx_v, row_v, colb_v, fetch_v, out_v, sem):
        wid = lax.axis_index("s") * NUM_CORES + lax.axis_index("c")
        pltpu.sync_copy(
            idx_hbm.at[pl.ds(wid * CHUNKS_PER_WORKER, CHUNKS_PER_WORKER)], idx_v
        )
        copies = []
        for j in range(CHUNKS_PER_WORKER):
            for s in range(CHUNK // 16):
                v = idx_v[j, pl.ds(s * 16, 16)]
                row_v[j, pl.ds(s * 16, 16)] = v >> 3
                colb_v[pl.ds(j * CHUNK + s * 16, 16)] = (v & 7) << 4
            copies.append(
                pltpu.async_copy(
                    table_hbm.at[row_v.at[j]],
                    fetch_v.at[pl.ds(j * CHUNK, CHUNK)],
                    sem,
                )
            )
        for c in copies:
            c.wait()

        lanes = lax.iota(jnp.int32, 16)

        def block(b, carry):
            rid = b * 16 + lanes
            colb = plsc.load_gather(colb_v, [rid])
            orow = 2 * b + (lanes >> 3)
            ocol0 = (lanes & 7) << 4
            cols = []
            ss = jnp.zeros((16,), jnp.float32)
            for c in range(EMBED_DIM):
                col = plsc.load_gather(fetch_v, [rid, colb + c])
                cols.append(col)
                ss = ss + col * col
            scale = jnp.where(ss > MAX_NORM * MAX_NORM, MAX_NORM * _rsqrt(ss), 1.0)
            for c in range(EMBED_DIM):
                plsc.store_scatter(out_v, [orow, ocol0 + c], cols[c] * scale)
            return carry

        lax.fori_loop(0, BLOCKS_PER_WORKER, block, None)

        pltpu.sync_copy(
            out_v, out_hbm.at[pl.ds(wid * OUT_STAGE_ROWS, OUT_STAGE_ROWS)]
        )

    return k(table2, idx2d)


def kernel(data, table):
    table2 = table.reshape(TAB_ROWS, 128)
    idx2d = data.reshape(NUM_WORKERS * CHUNKS_PER_WORKER, CHUNK)
    out = _sc_embed(table2, idx2d)
    return out.reshape(BATCH, EMBED_DIM)


# speedup vs baseline: 7.3460x; 7.3460x over previous
import functools

import jax
import jax.numpy as jnp
from jax import lax
from jax.experimental import pallas as pl
from jax.experimental.pallas import tpu as pltpu
from jax.experimental.pallas import tpu_sc as plsc

VOCAB_SIZE = 1000000
EMBED_DIM = 16
BATCH = 16384
MAX_NORM = 2.0

NUM_CORES = 2
NUM_SUBCORES = 16
NUM_WORKERS = NUM_CORES * NUM_SUBCORES
ROWS_PER_WORKER = BATCH // NUM_WORKERS
GROUP = 16
NUM_GROUPS = ROWS_PER_WORKER // GROUP
RING_GROUPS = 2
RING = RING_GROUPS * GROUP


def _rsqrt(x):
    i = lax.bitcast_convert_type(x, jnp.int32)
    y = lax.bitcast_convert_type(jnp.int32(0x5F3759DF) - (i >> 1), jnp.float32)
    for _ in range(3):
        y = y * (1.5 - 0.5 * x * y * y)
    return y


def _sc_embed(table_t, idx):
    mesh = plsc.VectorSubcoreMesh(core_axis_name="c", subcore_axis_name="s")

    @functools.partial(
        pl.kernel,
        out_type=jax.ShapeDtypeStruct((EMBED_DIM, BATCH), jnp.float32),
        mesh=mesh,
        compiler_params=pltpu.CompilerParams(needs_layout_passes=False),
        scratch_types=[
            pltpu.VMEM((ROWS_PER_WORKER,), jnp.int32),
            pltpu.VMEM((RING, EMBED_DIM, 128), jnp.float32),
            pltpu.VMEM((EMBED_DIM, ROWS_PER_WORKER), jnp.float32),
            pltpu.SemaphoreType.DMA,
        ],
    )
    def k(table_hbm, idx_hbm, out_hbm, idx_v, blocks_v, out_v, sem):
        wid = lax.axis_index("s") * NUM_CORES + lax.axis_index("c")
        base = wid * ROWS_PER_WORKER
        pltpu.sync_copy(idx_hbm.at[pl.ds(base, ROWS_PER_WORKER)], idx_v)

        def fire(g, slot):
            v = (idx_v[pl.ds(g * GROUP, GROUP)] >> 7) << 7
            for r16 in range(GROUP):
                cb = pl.multiple_of(v[r16], 128)
                pltpu.async_copy(
                    table_hbm.at[:, pl.ds(cb, 128)],
                    blocks_v.at[slot + r16],
                    sem,
                )

        lanes = lax.iota(jnp.int32, 16)

        def process(g, slot):
            for _ in range(GROUP):
                pltpu.make_async_copy(
                    table_hbm.at[:, pl.ds(0, 128)], blocks_v.at[0], sem
                ).wait()
            col = idx_v[pl.ds(g * GROUP, GROUP)] & 127
            bufs = slot + lanes
            vals = []
            for d in range(EMBED_DIM):
                vals.append(
                    plsc.load_gather(
                        blocks_v, [bufs, jnp.full((16,), d, jnp.int32), col]
                    )
                )
            ss = vals[0] * vals[0]
            for d in range(1, EMBED_DIM):
                ss = ss + vals[d] * vals[d]
            scale = jnp.where(ss > MAX_NORM * MAX_NORM, MAX_NORM * _rsqrt(ss), 1.0)
            osl = pl.ds(g * GROUP, GROUP)
            for d in range(EMBED_DIM):
                out_v[d, osl] = vals[d] * scale

        fire(0, 0)
        fire(1, GROUP)

        @pl.loop(0, NUM_GROUPS)
        def _pipeline(g):
            slot = (g % RING_GROUPS) * GROUP
            process(g, slot)

            @pl.when(g < NUM_GROUPS - RING_GROUPS)
            def _refill():
                fire(g + RING_GROUPS, slot)

        pltpu.sync_copy(out_v, out_hbm.at[:, pl.ds(base, ROWS_PER_WORKER)])

    return k(table_t, idx)


def kernel(data, table):
    table_t = table.T
    out_t = _sc_embed(table_t, data)
    return out_t.T
